# baseline (speedup 1.0000x reference)
import jax
import jax.numpy as jnp
from jax.experimental import pallas as pl
from jax.experimental.pallas import tpu as pltpu

_CH = 256
_NBUF = 8
_NPRI = 2


def _body(a_hbm, f_ref, o_ref, buf, sems):
    B, M, K = a_hbm.shape
    cpb = M // _CH
    total = B * cpb
    ngroups = total // _NBUF

    def copy(c, slot):
        b = c // cpb
        r = jax.lax.rem(c, cpb)
        return pltpu.make_async_copy(
            a_hbm.at[b, pl.ds(r * _CH, _CH), :],
            buf.at[slot],
            sems.at[slot],
        )

    for slot in range(_NBUF):
        copy(slot, slot).start(priority=slot % _NPRI)

    def group(g, carry):
        base = g * _NBUF
        for slot in range(_NBUF):
            c = base + slot
            copy(c, slot).wait()
            b = c // cpb
            o_ref[pl.ds(c * _CH, _CH), :] = jnp.dot(
                buf[slot], f_ref[b], preferred_element_type=jnp.float32)

            @pl.when(c + _NBUF < total)
            def _(c=c, slot=slot):
                copy(c + _NBUF, slot).start(priority=slot % _NPRI)

        return carry

    jax.lax.fori_loop(0, ngroups, group, 0)


def kernel(features, A):
    B, M, K = A.shape
    N = features.shape[-1]
    out_flat = pl.pallas_call(
        _body,
        in_specs=[
            pl.BlockSpec(memory_space=pltpu.MemorySpace.HBM),
            pl.BlockSpec(memory_space=pltpu.MemorySpace.VMEM),
        ],
        out_specs=pl.BlockSpec(memory_space=pltpu.MemorySpace.VMEM),
        out_shape=jax.ShapeDtypeStruct((B * M, N), jnp.float32),
        scratch_shapes=[
            pltpu.VMEM((_NBUF, _CH, K), jnp.float32),
            pltpu.SemaphoreType.DMA((_NBUF,)),
        ],
    )(A, features)
    return out_flat.reshape(B, M, N)
